# baseline (speedup 1.0000x reference)
import functools

import jax
import jax.numpy as jnp
from jax import lax
from jax.experimental import pallas as pl
from jax.experimental.pallas import tpu as pltpu
from jax.experimental.pallas import tpu_sc as plsc

_INFO = plsc.get_sparse_core_info()
_NC, _NS = _INFO.num_cores, _INFO.num_subcores
_NW = _NC * _NS

_SEQ = 200
_D = 128
_BPW = 16

_mesh = plsc.VectorSubcoreMesh(
    core_axis_name="c", subcore_axis_name="s", num_cores=1
)


@functools.partial(
    pl.kernel,
    mesh=_mesh,
    out_type=jax.ShapeDtypeStruct((_SEQ, 2 * _D), jnp.float32),
    scratch_types=[
        pltpu.VMEM((_BPW,), jnp.int32),
        pltpu.VMEM((_BPW, 2 * _D), jnp.float32),
        pltpu.SemaphoreType.DMA,
        pltpu.SemaphoreType.DMA,
        pltpu.SemaphoreType.DMA,
    ],
)
def _encode(idx_hbm, emb_hbm, pos_hbm, out_hbm, idx_v, comb_v, sem_i, sem_p, sem_g):
    wid = lax.axis_index("s")

    def _chunk(base, rows):
        idx_cp = pltpu.async_copy(
            idx_hbm.at[pl.ds(base, rows)], idx_v.at[pl.ds(0, rows)], sem_i
        )
        pos_cp = pltpu.async_copy(
            pos_hbm.at[pl.ds(base, rows)],
            comb_v.at[pl.ds(0, rows), pl.ds(_D, _D)],
            sem_p,
        )
        idx_cp.wait()
        gat = pltpu.async_copy(
            emb_hbm.at[idx_v.at[pl.ds(0, rows)]],
            comb_v.at[pl.ds(0, rows), pl.ds(0, _D)],
            sem_g,
        )
        pos_cp.wait()
        gat.wait()
        pltpu.sync_copy(
            comb_v.at[pl.ds(0, rows)], out_hbm.at[pl.ds(base, rows)]
        )

    @pl.when(wid < _SEQ // _BPW)
    def _():
        _chunk(wid * _BPW, _BPW)

    @pl.when(wid == _SEQ // _BPW)
    def _():
        _chunk((_SEQ // _BPW) * _BPW, _SEQ % _BPW)


def kernel(fnums, emb_table, pos_table):
    idx = fnums.astype(jnp.int32)
    return _encode(idx, emb_table, pos_table)


# speedup vs baseline: 1.0105x; 1.0105x over previous
import functools

import jax
import jax.numpy as jnp
from jax import lax
from jax.experimental import pallas as pl
from jax.experimental.pallas import tpu as pltpu
from jax.experimental.pallas import tpu_sc as plsc

_INFO = plsc.get_sparse_core_info()
_NC, _NS = _INFO.num_cores, _INFO.num_subcores
_NW = _NC * _NS

_SEQ = 200
_D = 128
_BPW = 16

_mesh = plsc.VectorSubcoreMesh(
    core_axis_name="c", subcore_axis_name="s", num_cores=1
)


@functools.partial(
    pl.kernel,
    mesh=_mesh,
    out_type=jax.ShapeDtypeStruct((_SEQ, 2 * _D), jnp.float32),
    scratch_types=[
        pltpu.VMEM((_BPW,), jnp.int32),
        pltpu.VMEM((_BPW, 2 * _D), jnp.float32),
        pltpu.SemaphoreType.DMA,
        pltpu.SemaphoreType.DMA,
        pltpu.SemaphoreType.DMA,
    ],
)
def _encode(idx_hbm, emb_hbm, pos_hbm, out_hbm, idx_v, comb_v, sem_i, sem_p, sem_g):
    wid = lax.axis_index("s")
    base = jnp.minimum(wid * _BPW, _SEQ - _BPW)
    idx_cp = pltpu.async_copy(idx_hbm.at[pl.ds(base, _BPW)], idx_v, sem_i)
    pos_cp = pltpu.async_copy(
        pos_hbm.at[pl.ds(base, _BPW)], comb_v.at[:, pl.ds(_D, _D)], sem_p
    )
    idx_cp.wait()
    gat = pltpu.async_copy(emb_hbm.at[idx_v], comb_v.at[:, pl.ds(0, _D)], sem_g)
    pos_cp.wait()
    gat.wait()
    pltpu.sync_copy(comb_v, out_hbm.at[pl.ds(base, _BPW)])


def kernel(fnums, emb_table, pos_table):
    idx = fnums.astype(jnp.int32)
    return _encode(idx, emb_table, pos_table)


# speedup vs baseline: 1.0136x; 1.0031x over previous
import functools

import jax
import jax.numpy as jnp
from jax import lax
from jax.experimental import pallas as pl
from jax.experimental.pallas import tpu as pltpu
from jax.experimental.pallas import tpu_sc as plsc

_INFO = plsc.get_sparse_core_info()
_NC, _NS = _INFO.num_cores, _INFO.num_subcores
_NW = _NC * _NS

_SEQ = 200
_D = 128
_BPW = 16

_NACT = (_SEQ + _BPW - 1) // _BPW
_mesh = plsc.VectorSubcoreMesh(
    core_axis_name="c", subcore_axis_name="s", num_cores=1, num_subcores=_NACT
)


@functools.partial(
    pl.kernel,
    mesh=_mesh,
    out_type=jax.ShapeDtypeStruct((_SEQ, 2 * _D), jnp.float32),
    scratch_types=[
        pltpu.VMEM((_BPW,), jnp.int32),
        pltpu.VMEM((_BPW, 2 * _D), jnp.float32),
        pltpu.SemaphoreType.DMA,
        pltpu.SemaphoreType.DMA,
        pltpu.SemaphoreType.DMA,
    ],
)
def _encode(idx_hbm, emb_hbm, pos_hbm, out_hbm, idx_v, comb_v, sem_i, sem_p, sem_g):
    wid = lax.axis_index("s")
    base = jnp.minimum(wid * _BPW, _SEQ - _BPW)
    idx_cp = pltpu.async_copy(idx_hbm.at[pl.ds(base, _BPW)], idx_v, sem_i)
    pos_cp = pltpu.async_copy(
        pos_hbm.at[pl.ds(base, _BPW)], comb_v.at[:, pl.ds(_D, _D)], sem_p
    )
    idx_cp.wait()
    gat = pltpu.async_copy(emb_hbm.at[idx_v], comb_v.at[:, pl.ds(0, _D)], sem_g)
    pos_cp.wait()
    gat.wait()
    pltpu.sync_copy(comb_v, out_hbm.at[pl.ds(base, _BPW)])


def kernel(fnums, emb_table, pos_table):
    idx = fnums.astype(jnp.int32)
    return _encode(idx, emb_table, pos_table)


# speedup vs baseline: 1.0333x; 1.0194x over previous
import functools

import jax
import jax.numpy as jnp
from jax import lax
from jax.experimental import pallas as pl
from jax.experimental.pallas import tpu as pltpu
from jax.experimental.pallas import tpu_sc as plsc

_SEQ = 200
_D = 128
_BPW = 16
_NACT = (_SEQ + _BPW - 1) // _BPW
_PAD = _NACT * _BPW

_smesh = plsc.ScalarSubcoreMesh(axis_name="c", num_cores=1)
_vmesh = plsc.VectorSubcoreMesh(
    core_axis_name="c", subcore_axis_name="s", num_cores=1, num_subcores=_NACT
)


def _scs_body(idx_hbm, emb_hbm, pos_hbm, out_hbm, idx_sh, rdy, idx_v, comb_v,
              sem_p, sem_g):
    del emb_hbm, pos_hbm, out_hbm, idx_v, comb_v, sem_p, sem_g
    pltpu.sync_copy(idx_hbm, idx_sh)
    for i in range(_NACT):
        pltpu.semaphore_signal(rdy, 1, device_id={"s": i})


def _tec_body(idx_hbm, emb_hbm, pos_hbm, out_hbm, idx_sh, rdy, idx_v, comb_v,
              sem_p, sem_g):
    del idx_hbm
    wid = lax.axis_index("s")
    base = jnp.minimum(wid * _BPW, _SEQ - _BPW)
    pos_cp = pltpu.async_copy(
        pos_hbm.at[pl.ds(base, _BPW)], comb_v.at[:, pl.ds(_D, _D)], sem_p
    )
    pltpu.semaphore_wait(rdy, 1)
    pltpu.sync_copy(idx_sh.at[pl.ds(base, _BPW)], idx_v)
    gat = pltpu.async_copy(emb_hbm.at[idx_v], comb_v.at[:, pl.ds(0, _D)], sem_g)
    pos_cp.wait()
    gat.wait()
    pltpu.sync_copy(comb_v, out_hbm.at[pl.ds(base, _BPW)])


_encode = pl.kernel(
    [_scs_body, _tec_body],
    out_type=jax.ShapeDtypeStruct((_SEQ, 2 * _D), jnp.float32),
    mesh=[_smesh, _vmesh],
    scratch_types=[
        pltpu.MemorySpace.VMEM_SHARED((_SEQ,), jnp.int32),
        pltpu.SemaphoreType.REGULAR @ _vmesh,
        pltpu.VMEM((_BPW,), jnp.int32) @ _vmesh,
        pltpu.VMEM((_BPW, 2 * _D), jnp.float32) @ _vmesh,
        pltpu.SemaphoreType.DMA @ _vmesh,
        pltpu.SemaphoreType.DMA @ _vmesh,
    ],
)


def kernel(fnums, emb_table, pos_table):
    idx = fnums.astype(jnp.int32)
    return _encode(idx, emb_table, pos_table)
